# baseline (speedup 1.0000x reference)
import jax
import jax.numpy as jnp
from jax.experimental import pallas as pl
from jax.experimental.pallas import tpu as pltpu

B, H, S, E = 8, 16, 2048, 128
_HB = 8


def _copy_one(n_ref, t_ref, cache_ref, out_ref):
    out_ref[...] = cache_ref[...]
    slot = n_ref[0] + 1
    out_ref[0, :, pl.ds(slot, 1), :] = t_ref[0, :, :, :]


def _copy_call(n_arr, t, cache):
    cache_spec = pl.BlockSpec((1, _HB, S, E), lambda b, h: (b, h, 0, 0))
    t_spec = pl.BlockSpec((1, _HB, 1, E), lambda b, h: (b, h, 0, 0))
    return pl.pallas_call(
        _copy_one,
        grid=(B, H // _HB),
        out_shape=jax.ShapeDtypeStruct(cache.shape, cache.dtype),
        in_specs=[pl.BlockSpec(memory_space=pltpu.MemorySpace.SMEM),
                  t_spec, cache_spec],
        out_specs=cache_spec,
        compiler_params=pltpu.CompilerParams(
            dimension_semantics=("parallel", "parallel")),
    )(n_arr, t, cache)


def kernel(k_t, v_t, k_cache, v_cache, n_cached):
    n_arr = jnp.asarray(n_cached, jnp.int32).reshape(1)
    new_k = _copy_call(n_arr, k_t, k_cache)
    new_v = _copy_call(n_arr, v_t, v_cache)
    return (new_k, new_v)
